# baseline (speedup 1.0000x reference)
import numpy as np
import jax
import jax.numpy as jnp
from jax.experimental import pallas as pl
from jax.experimental.pallas import tpu as pltpu

_ALPHA = 4


def kernel(frames):
    C, T, H, W = frames.shape
    HW = H * W
    Ts = T // _ALPHA
    idx = np.linspace(0, T - 1, Ts).astype(np.int32)

    f = frames.reshape(C, T, HW)

    def body(in_ref, slow_ref, fast_ref, sem_fast, sem_slow):
        fast_copy = pltpu.make_async_copy(in_ref, fast_ref, sem_fast)
        fast_copy.start()
        copies = []
        for c in range(C):
            for p, g in enumerate(idx):
                cp = pltpu.make_async_copy(
                    in_ref.at[c, int(g)], slow_ref.at[c, p], sem_slow)
                cp.start()
                copies.append(cp)
        for cp in copies:
            cp.wait()
        fast_copy.wait()

    slow3, fast3 = pl.pallas_call(
        body,
        in_specs=[pl.BlockSpec(memory_space=pl.ANY)],
        out_specs=[
            pl.BlockSpec(memory_space=pl.ANY),
            pl.BlockSpec(memory_space=pl.ANY),
        ],
        out_shape=[
            jax.ShapeDtypeStruct((C, Ts, HW), frames.dtype),
            jax.ShapeDtypeStruct((C, T, HW), frames.dtype),
        ],
        scratch_shapes=[pltpu.SemaphoreType.DMA, pltpu.SemaphoreType.DMA],
    )(f)

    return (slow3.reshape(C, Ts, H, W), fast3.reshape(C, T, H, W))


# speedup vs baseline: 12.2924x; 12.2924x over previous
import numpy as np
import jax
import jax.numpy as jnp
from jax.experimental import pallas as pl
from jax.experimental.pallas import tpu as pltpu

_ALPHA = 4
_CHUNK = 8


def kernel(frames):
    C, T, H, W = frames.shape
    HW = H * W
    Ts = T // _ALPHA
    idx = np.linspace(0, T - 1, Ts).astype(np.int32)
    nj = T // _CHUNK

    f = frames.reshape(C, T, HW)

    def body(in_ref, slow_ref, fast_ref, buf, sin, sout):
        ins = []
        n = 0
        for c in range(C):
            for j in range(nj):
                sl = pl.ds(j * _CHUNK, _CHUNK)
                cp = pltpu.make_async_copy(
                    in_ref.at[c, sl], buf.at[c, sl], sin.at[n])
                cp.start()
                ins.append((c, j, cp))
                n += 1
        outs = []
        for c, j, cp in ins:
            cp.wait()
            sl = pl.ds(j * _CHUNK, _CHUNK)
            o = pltpu.make_async_copy(
                buf.at[c, sl], fast_ref.at[c, sl], sout)
            o.start()
            outs.append(o)
            lo, hi = j * _CHUNK, (j + 1) * _CHUNK
            for p, g in enumerate(idx):
                if lo <= g < hi:
                    o2 = pltpu.make_async_copy(
                        buf.at[c, int(g)], slow_ref.at[c, int(p)], sout)
                    o2.start()
                    outs.append(o2)
        for o in outs:
            o.wait()

    slow3, fast3 = pl.pallas_call(
        body,
        in_specs=[pl.BlockSpec(memory_space=pl.ANY)],
        out_specs=[
            pl.BlockSpec(memory_space=pl.ANY),
            pl.BlockSpec(memory_space=pl.ANY),
        ],
        out_shape=[
            jax.ShapeDtypeStruct((C, Ts, HW), frames.dtype),
            jax.ShapeDtypeStruct((C, T, HW), frames.dtype),
        ],
        scratch_shapes=[
            pltpu.VMEM((C, T, HW), frames.dtype),
            pltpu.SemaphoreType.DMA((C * nj,)),
            pltpu.SemaphoreType.DMA,
        ],
    )(f)

    return (slow3.reshape(C, Ts, H, W), fast3.reshape(C, T, H, W))
